# baseline (speedup 1.0000x reference)
import functools

import jax
import jax.numpy as jnp
from jax import lax
from jax.experimental import pallas as pl
from jax.experimental.pallas import tpu as pltpu
from jax.experimental.pallas import tpu_sc as plsc

_N = 16384
_H = 128
_NV = 8
_NAME_K = 20
_DESC_K = 50
_TYPE_K = 20
_NW = 32
_RPW = _N // _NW
_CH = 64
_NCH = _RPW // _CH
_EPS = 1e-12


def _lane_extract(vec16, lane):
    m = lax.iota(jnp.int32, 16) == jnp.full((16,), lane, jnp.int32)
    return jnp.sum(jnp.where(m, vec16, 0.0))


def _rsqrt_scalar(v):
    i = lax.bitcast_convert_type(v, jnp.int32)
    i = jnp.int32(0x5F3759DF) - lax.shift_right_arithmetic(i, 1)
    y = lax.bitcast_convert_type(i, jnp.float32)
    for _ in range(3):
        y = y * (1.5 - 0.5 * v * y * y)
    return y


def _recip_scalar(v):
    y = _rsqrt_scalar(v)
    return y * y


def _ln_write(pool_p, r, inv_len, lnw_v, lnb_v, out_p, col):
    x = [pool_p[r, pl.ds(j * 16, 16)] * inv_len for j in range(_NV)]
    s = x[0]
    q = x[0] * x[0]
    for j in range(1, _NV):
        s = s + x[j]
        q = q + x[j] * x[j]
    mu = jnp.sum(s) * (1.0 / _H)
    var = jnp.maximum(jnp.sum(q) * (1.0 / _H) - mu * mu, 0.0)
    inv_std = _rsqrt_scalar(var + _EPS)
    for j in range(_NV):
        w = lnw_v[pl.ds(j * 16, 16)]
        b = lnb_v[pl.ds(j * 16, 16)]
        out_p[r, pl.ds(col + j * 16, 16)] = (x[j] - mu) * inv_std * w + b


def _make_kernel():
    mesh = plsc.VectorSubcoreMesh(core_axis_name="c", subcore_axis_name="s")

    @functools.partial(
        pl.kernel,
        mesh=mesh,
        compiler_params=pltpu.CompilerParams(needs_layout_passes=False),
        out_type=jax.ShapeDtypeStruct((_N, 3 * _H), jnp.float32),
        scratch_types=[
            pltpu.VMEM((2, _NAME_K, _CH), jnp.int32),
            pltpu.VMEM((2, _DESC_K, _CH), jnp.int32),
            pltpu.VMEM((2, _TYPE_K, _CH), jnp.int32),
            pltpu.VMEM((2, _CH, _H), jnp.float32),
            pltpu.VMEM((2, _CH, _H), jnp.float32),
            pltpu.VMEM((2, _CH, _H), jnp.float32),
            pltpu.VMEM((2, _CH, 3 * _H), jnp.float32),
            pltpu.VMEM((_H,), jnp.float32),
            pltpu.VMEM((_H,), jnp.float32),
            pltpu.VMEM((_RPW,), jnp.float32),
            pltpu.VMEM((_RPW,), jnp.float32),
            pltpu.VMEM((_RPW,), jnp.float32),
            pltpu.SemaphoreType.DMA,
            pltpu.SemaphoreType.DMA,
            pltpu.SemaphoreType.DMA,
            pltpu.SemaphoreType.DMA,
            pltpu.SemaphoreType.DMA,
        ],
    )
    def emb_kernel(nidx_hbm, didx_hbm, tidx_hbm, nlen_hbm, dlen_hbm,
                   tlen_hbm, wemb_hbm, temb_hbm, lnw_hbm, lnb_hbm, out_hbm,
                   nidx_v, didx_v, tidx_v, npool_v, dpool_v, tpool_v,
                   out_v, lnw_v, lnb_v, nlen_v, dlen_v, tlen_v,
                   sem_a0, sem_a1, sem_i, sem_o0, sem_o1):
        wid = lax.axis_index("s") * 2 + lax.axis_index("c")
        base = wid * _RPW
        cbase = wid * _NCH
        sem_a = (sem_a0, sem_a1)
        sem_o = (sem_o0, sem_o1)
        pltpu.sync_copy(lnw_hbm, lnw_v)
        pltpu.sync_copy(lnb_hbm, lnb_v)
        pltpu.sync_copy(nlen_hbm.at[pl.ds(base, _RPW)], nlen_v)
        pltpu.sync_copy(dlen_hbm.at[pl.ds(base, _RPW)], dlen_v)
        pltpu.sync_copy(tlen_hbm.at[pl.ds(base, _RPW)], tlen_v)

        fields = ((nidx_v, nidx_hbm, npool_v, wemb_hbm, _NAME_K),
                  (didx_v, didx_hbm, dpool_v, wemb_hbm, _DESC_K),
                  (tidx_v, tidx_hbm, tpool_v, temb_hbm, _TYPE_K))

        def idx_descs(c, p):
            return [pltpu.make_async_copy(ihbm.at[cbase + c], iv.at[p],
                                          sem_i)
                    for iv, ihbm, _, _, _ in fields]

        def issue_idx(c, p):
            for d in idx_descs(c, p):
                d.start()

        def wait_idx(c, p):
            for d in idx_descs(c, p):
                d.wait()

        def zero_pools(p):
            zero = jnp.zeros((16,), jnp.float32)

            def zbody(r, _):
                for pool in (npool_v, dpool_v, tpool_v):
                    pp = pool.at[p]
                    for j in range(_NV):
                        pp[r, pl.ds(j * 16, 16)] = zero
                return 0

            lax.fori_loop(0, _CH, zbody, 0)

        def issue_adds(p):
            for iv, _, pool, table, kk in fields:
                def abody(k, _):
                    pltpu.async_copy(table.at[iv.at[p].at[k]], pool.at[p],
                                     sem_a[p], add=True)
                    return 0

                lax.fori_loop(0, kk, abody, 0)

        def wait_adds(p):
            for iv, _, pool, table, kk in fields:
                def wbody(k, _):
                    pltpu.make_async_copy(table.at[iv.at[p].at[k]],
                                          pool.at[p], sem_a[p]).wait()
                    return 0

                lax.fori_loop(0, kk, wbody, 0)

        def out_desc(c, p):
            return pltpu.make_async_copy(
                out_v.at[p], out_hbm.at[pl.ds(base + c * _CH, _CH)],
                sem_o[p])

        def ln_chunk(c, p):
            np_, dp_, tp_ = npool_v.at[p], dpool_v.at[p], tpool_v.at[p]
            op_ = out_v.at[p]

            def lbody(r, _):
                gr = c * _CH + r
                rb = pl.multiple_of((gr >> 4) << 4, 16)
                lane = gr & 15
                inv_n = _recip_scalar(
                    _lane_extract(nlen_v[pl.ds(rb, 16)], lane))
                inv_d = _recip_scalar(
                    _lane_extract(dlen_v[pl.ds(rb, 16)], lane))
                inv_t = _recip_scalar(
                    _lane_extract(tlen_v[pl.ds(rb, 16)], lane))
                _ln_write(np_, r, inv_n, lnw_v, lnb_v, op_, 0)
                _ln_write(dp_, r, inv_d, lnw_v, lnb_v, op_, _H)
                _ln_write(tp_, r, inv_t, lnw_v, lnb_v, op_, 2 * _H)
                return 0

            lax.fori_loop(0, _CH, lbody, 0)

        issue_idx(0, 0)
        wait_idx(0, 0)
        zero_pools(0)
        zero_pools(1)
        issue_idx(1, 1)
        issue_adds(0)

        for c in range(_NCH):
            p = c & 1
            q = 1 - p
            wait_adds(p)
            if c + 1 < _NCH:
                wait_idx(c + 1, q)
                if c + 2 < _NCH:
                    issue_idx(c + 2, p)
                issue_adds(q)
            if c >= 2:
                out_desc(c - 2, p).wait()
            ln_chunk(c, p)
            out_desc(c, p).start()
            zero_pools(p)

        out_desc(_NCH - 2, 0).wait()
        out_desc(_NCH - 1, 1).wait()

    return emb_kernel


_EMB_KERNEL = _make_kernel()


def kernel(cand_name, cand_name_length, cand_description,
           cand_description_length, cand_type, cand_type_length,
           word_emb, ent_type_emb, ln_w, ln_b):
    def t(a, k):
        return (a.reshape(_N // _CH, _CH, k).transpose(0, 2, 1)
                .astype(jnp.int32))

    out = _EMB_KERNEL(t(cand_name, _NAME_K),
                      t(cand_description, _DESC_K),
                      t(cand_type, _TYPE_K),
                      cand_name_length.reshape(_N),
                      cand_description_length.reshape(_N),
                      cand_type_length.reshape(_N),
                      word_emb, ent_type_emb, ln_w, ln_b)
    return out.reshape(1024, 16, 3 * _H)


# speedup vs baseline: 1.3362x; 1.3362x over previous
import functools

import jax
import jax.numpy as jnp
from jax import lax
from jax.experimental import pallas as pl
from jax.experimental.pallas import tpu as pltpu
from jax.experimental.pallas import tpu_sc as plsc

_N = 16384
_H = 128
_NV = 8
_NAME_K = 20
_DESC_K = 50
_TYPE_K = 20
_TVOCAB = 1000
_NW = 32
_RPW = _N // _NW
_CH = 64
_NCH = _RPW // _CH
_BLK = 512
_EPS = 1e-12


def _lane_extract(vec16, lane):
    m = lax.iota(jnp.int32, 16) == jnp.full((16,), lane, jnp.int32)
    return jnp.sum(jnp.where(m, vec16, 0.0))


def _rsqrt_scalar(v):
    i = lax.bitcast_convert_type(v, jnp.int32)
    i = jnp.int32(0x5F3759DF) - lax.shift_right_arithmetic(i, 1)
    y = lax.bitcast_convert_type(i, jnp.float32)
    for _ in range(3):
        y = y * (1.5 - 0.5 * v * y * y)
    return y


def _recip_scalar(v):
    y = _rsqrt_scalar(v)
    return y * y


def _ln_write(pool_p, r, inv_len, lnw_v, lnb_v, out_p, col):
    x = [pool_p[r, pl.ds(j * 16, 16)] * inv_len for j in range(_NV)]
    s = x[0]
    q = x[0] * x[0]
    for j in range(1, _NV):
        s = s + x[j]
        q = q + x[j] * x[j]
    mu = jnp.sum(s) * (1.0 / _H)
    var = jnp.maximum(jnp.sum(q) * (1.0 / _H) - mu * mu, 0.0)
    inv_std = _rsqrt_scalar(var + _EPS)
    for j in range(_NV):
        w = lnw_v[pl.ds(j * 16, 16)]
        b = lnb_v[pl.ds(j * 16, 16)]
        out_p[r, pl.ds(col + j * 16, 16)] = (x[j] - mu) * inv_std * w + b


def _make_sc_kernel():
    mesh = plsc.VectorSubcoreMesh(core_axis_name="c", subcore_axis_name="s")

    @functools.partial(
        pl.kernel,
        mesh=mesh,
        compiler_params=pltpu.CompilerParams(needs_layout_passes=False),
        out_type=jax.ShapeDtypeStruct((_N, 2 * _H), jnp.float32),
        scratch_types=[
            pltpu.VMEM((2, _NAME_K, _CH), jnp.int32),
            pltpu.VMEM((2, _DESC_K, _CH), jnp.int32),
            pltpu.VMEM((2, _CH, _H), jnp.float32),
            pltpu.VMEM((2, _CH, _H), jnp.float32),
            pltpu.VMEM((2, _CH, 2 * _H), jnp.float32),
            pltpu.VMEM((_H,), jnp.float32),
            pltpu.VMEM((_H,), jnp.float32),
            pltpu.VMEM((_RPW,), jnp.float32),
            pltpu.VMEM((_RPW,), jnp.float32),
            pltpu.SemaphoreType.DMA,
            pltpu.SemaphoreType.DMA,
            pltpu.SemaphoreType.DMA,
            pltpu.SemaphoreType.DMA,
            pltpu.SemaphoreType.DMA,
        ],
    )
    def emb_kernel(nidx_hbm, didx_hbm, nlen_hbm, dlen_hbm,
                   wemb_hbm, lnw_hbm, lnb_hbm, out_hbm,
                   nidx_v, didx_v, npool_v, dpool_v,
                   out_v, lnw_v, lnb_v, nlen_v, dlen_v,
                   sem_a0, sem_a1, sem_i, sem_o0, sem_o1):
        wid = lax.axis_index("s") * 2 + lax.axis_index("c")
        base = wid * _RPW
        cbase = wid * _NCH
        sem_a = (sem_a0, sem_a1)
        sem_o = (sem_o0, sem_o1)
        pltpu.sync_copy(lnw_hbm, lnw_v)
        pltpu.sync_copy(lnb_hbm, lnb_v)
        pltpu.sync_copy(nlen_hbm.at[pl.ds(base, _RPW)], nlen_v)
        pltpu.sync_copy(dlen_hbm.at[pl.ds(base, _RPW)], dlen_v)

        fields = ((nidx_v, nidx_hbm, npool_v, _NAME_K),
                  (didx_v, didx_hbm, dpool_v, _DESC_K))

        def idx_descs(c, p):
            return [pltpu.make_async_copy(ihbm.at[cbase + c], iv.at[p],
                                          sem_i)
                    for iv, ihbm, _, _ in fields]

        def issue_idx(c, p):
            for d in idx_descs(c, p):
                d.start()

        def wait_idx(c, p):
            for d in idx_descs(c, p):
                d.wait()

        def zero_pools(p):
            zero = jnp.zeros((16,), jnp.float32)

            def zbody(r, _):
                for pool in (npool_v, dpool_v):
                    pp = pool.at[p]
                    for j in range(_NV):
                        pp[r, pl.ds(j * 16, 16)] = zero
                return 0

            lax.fori_loop(0, _CH, zbody, 0)

        def issue_adds(p):
            for iv, _, pool, kk in fields:
                def abody(k, _):
                    pltpu.async_copy(wemb_hbm.at[iv.at[p].at[k]],
                                     pool.at[p], sem_a[p], add=True)
                    return 0

                lax.fori_loop(0, kk, abody, 0)

        def wait_adds(p):
            for iv, _, pool, kk in fields:
                def wbody(k, _):
                    pltpu.make_async_copy(wemb_hbm.at[iv.at[p].at[k]],
                                          pool.at[p], sem_a[p]).wait()
                    return 0

                lax.fori_loop(0, kk, wbody, 0)

        def out_desc(c, p):
            return pltpu.make_async_copy(
                out_v.at[p], out_hbm.at[pl.ds(base + c * _CH, _CH)],
                sem_o[p])

        def ln_chunk(c, p):
            np_, dp_ = npool_v.at[p], dpool_v.at[p]
            op_ = out_v.at[p]

            def lbody(r, _):
                gr = c * _CH + r
                rb = pl.multiple_of((gr >> 4) << 4, 16)
                lane = gr & 15
                inv_n = _recip_scalar(
                    _lane_extract(nlen_v[pl.ds(rb, 16)], lane))
                inv_d = _recip_scalar(
                    _lane_extract(dlen_v[pl.ds(rb, 16)], lane))
                _ln_write(np_, r, inv_n, lnw_v, lnb_v, op_, 0)
                _ln_write(dp_, r, inv_d, lnw_v, lnb_v, op_, _H)
                return 0

            lax.fori_loop(0, _CH, lbody, 0)

        issue_idx(0, 0)
        wait_idx(0, 0)
        zero_pools(0)
        zero_pools(1)
        issue_idx(1, 1)
        issue_adds(0)

        for c in range(_NCH):
            p = c & 1
            q = 1 - p
            wait_adds(p)
            if c + 1 < _NCH:
                wait_idx(c + 1, q)
                if c + 2 < _NCH:
                    issue_idx(c + 2, p)
                issue_adds(q)
            if c >= 2:
                out_desc(c - 2, p).wait()
            ln_chunk(c, p)
            out_desc(c, p).start()
            zero_pools(p)

        out_desc(_NCH - 2, 0).wait()
        out_desc(_NCH - 1, 1).wait()

    return emb_kernel


_SC_KERNEL = _make_sc_kernel()


def _tc_type_body(idx_ref, tab_ref, len_ref, w_ref, b_ref, o_ref):
    idx = idx_ref[...]
    vocab_iota = lax.broadcasted_iota(jnp.int32, (_BLK, _TVOCAB), 1)
    counts = jnp.zeros((_BLK, _TVOCAB), jnp.float32)
    for k in range(_TYPE_K):
        counts += (idx[:, k][:, None] == vocab_iota).astype(jnp.float32)
    pooled = jnp.dot(counts, tab_ref[...],
                     preferred_element_type=jnp.float32)
    x = pooled / len_ref[...]
    mu = jnp.mean(x, axis=-1, keepdims=True)
    var = jnp.mean((x - mu) ** 2, axis=-1, keepdims=True)
    o_ref[...] = ((x - mu) * lax.rsqrt(var + _EPS) * w_ref[...]
                  + b_ref[...])


def _tc_type(tidx, temb, tlen, lnw, lnb):
    return pl.pallas_call(
        _tc_type_body,
        grid=(_N // _BLK,),
        in_specs=[
            pl.BlockSpec((_BLK, _TYPE_K), lambda i: (i, 0)),
            pl.BlockSpec((_TVOCAB, _H), lambda i: (0, 0)),
            pl.BlockSpec((_BLK, 1), lambda i: (i, 0)),
            pl.BlockSpec((1, _H), lambda i: (0, 0)),
            pl.BlockSpec((1, _H), lambda i: (0, 0)),
        ],
        out_specs=pl.BlockSpec((_BLK, _H), lambda i: (i, 0)),
        out_shape=jax.ShapeDtypeStruct((_N, _H), jnp.float32),
    )(tidx, temb, tlen, lnw, lnb)


def kernel(cand_name, cand_name_length, cand_description,
           cand_description_length, cand_type, cand_type_length,
           word_emb, ent_type_emb, ln_w, ln_b):
    def t(a, k):
        return (a.reshape(_N // _CH, _CH, k).transpose(0, 2, 1)
                .astype(jnp.int32))

    sc_out = _SC_KERNEL(t(cand_name, _NAME_K),
                        t(cand_description, _DESC_K),
                        cand_name_length.reshape(_N),
                        cand_description_length.reshape(_N),
                        word_emb, ln_w, ln_b)
    tc_out = _tc_type(cand_type.reshape(_N, _TYPE_K).astype(jnp.int32),
                      ent_type_emb,
                      cand_type_length.reshape(_N, 1),
                      ln_w.reshape(1, _H), ln_b.reshape(1, _H))
    out = jnp.concatenate([sc_out, tc_out], axis=-1)
    return out.reshape(1024, 16, 3 * _H)
